# baseline (speedup 1.0000x reference)
import functools

import jax
import jax.numpy as jnp
from jax.experimental import pallas as pl
from jax.experimental.pallas import tpu as pltpu

TB = 128


def _router_body(h_ref, rw_ref, pwt_ref, bexp_ref, *, T, E, TB_, NBMAX):
    f32 = jnp.float32
    h = h_ref[...]
    rw = rw_ref[...]
    logits = jax.lax.dot_general(h.astype(jnp.bfloat16),
                                 rw.astype(jnp.bfloat16),
                                 (((1,), (1,)), ((), ())),
                                 preferred_element_type=f32)
    lane = jax.lax.broadcasted_iota(jnp.int32, (T, E), 1).astype(f32)
    m1 = jnp.max(logits, axis=1, keepdims=True)
    i1 = jnp.min(jnp.where(logits == m1, lane, float(E)), axis=1, keepdims=True)
    e0 = (lane == i1).astype(f32)
    neg = jnp.where(lane == i1, -jnp.inf, logits)
    m2 = jnp.max(neg, axis=1, keepdims=True)
    i2 = jnp.min(jnp.where(neg == m2, lane, float(E)), axis=1, keepdims=True)
    e1 = (lane == i2).astype(f32)
    w0 = 1.0 / (1.0 + jnp.exp(m2 - m1))
    w1 = 1.0 - w0

    ones_col = jnp.ones((T, 1), f32)
    ecnt = jax.lax.dot_general(e0 + e1, ones_col, (((0,), (0,)), ((), ())),
                               preferred_element_type=f32,
                               precision=jax.lax.Precision.HIGHEST)
    nblk = jnp.floor((ecnt + (TB_ - 1)) / TB_)
    tri = (jax.lax.broadcasted_iota(jnp.int32, (E, E), 1)
           < jax.lax.broadcasted_iota(jnp.int32, (E, E), 0)).astype(f32)
    excl = jax.lax.dot_general(tri, nblk, (((1,), (0,)), ((), ())),
                               preferred_element_type=f32,
                               precision=jax.lax.Precision.HIGHEST)
    incl = excl + nblk
    off = excl * TB_

    Epairs = e0 + e1
    RB = 256
    c_blocks = []
    for tb in range(T // RB):
        r = jax.lax.broadcasted_iota(jnp.int32, (RB, T), 0) + tb * RB
        c = jax.lax.broadcasted_iota(jnp.int32, (RB, T), 1)
        lt = (c < r).astype(f32)
        c_blocks.append(jax.lax.dot_general(
            lt, Epairs, (((1,), (0,)), ((), ())),
            preferred_element_type=f32,
            precision=jax.lax.Precision.HIGHEST))
    C = jnp.concatenate(c_blocks, axis=0)

    off0 = jax.lax.dot_general(e0, off, (((1,), (0,)), ((), ())),
                               preferred_element_type=f32,
                               precision=jax.lax.Precision.HIGHEST)
    off1 = jax.lax.dot_general(e1, off, (((1,), (0,)), ((), ())),
                               preferred_element_type=f32,
                               precision=jax.lax.Precision.HIGHEST)
    rank0 = jnp.sum(C * e0, axis=1, keepdims=True)
    rank1 = jnp.sum(C * e1, axis=1, keepdims=True)
    pos0 = off0 + rank0
    pos1 = off1 + rank1

    cols = jnp.concatenate([pos0, pos1, w0, w1, w0, w0, w0, w0], axis=1)
    ident = (jax.lax.broadcasted_iota(jnp.int32, (T, T), 0)
             == jax.lax.broadcasted_iota(jnp.int32, (T, T), 1)).astype(f32)
    pwt_ref[...] = jax.lax.dot_general(cols, ident, (((0,), (0,)), ((), ())),
                                       preferred_element_type=f32,
                                       precision=jax.lax.Precision.HIGHEST)

    bidx = jax.lax.broadcasted_iota(jnp.int32, (E, NBMAX), 1).astype(f32)
    bexp = jnp.sum((incl <= bidx).astype(f32), axis=0, keepdims=True)
    bexp = jnp.minimum(bexp, float(E - 1))
    nused = jnp.sum(nblk)
    nrow = jnp.full((1, NBMAX), nused, f32)
    bexp_ref[...] = jnp.concatenate(
        [bexp, nrow, bexp, bexp, bexp, bexp, bexp, bexp], axis=0
    ).astype(jnp.int32)


def _ffn_body(bexp_sref, num_sref, pwt_ref, h_ref, gu_ref, dn_ref, out_ref,
              acc_ref, *, T, F, TB_, NBMAX):
    f32 = jnp.float32
    b = pl.program_id(0)

    @pl.when(b == 0)
    def _init():
        acc_ref[...] = jnp.zeros_like(acc_ref)

    @pl.when(b < num_sref[0])
    def _compute():
        pwt = pwt_ref[...]
        p0 = pwt[0:1, :]
        p1 = pwt[1:2, :]
        w0 = pwt[2:3, :]
        w1 = pwt[3:4, :]
        slot = (jax.lax.broadcasted_iota(jnp.int32, (TB_, T), 0)
                + b * TB_).astype(f32)
        m0 = (slot == p0).astype(f32)
        m1 = (slot == p1).astype(f32)
        Mg = m0 + m1
        Ms = m0 * w0 + m1 * w1
        x = jax.lax.dot_general(Mg, h_ref[...], (((1,), (0,)), ((), ())),
                                preferred_element_type=f32,
                                precision=jax.lax.Precision.HIGHEST)
        gu = gu_ref[0]
        gate = jax.lax.dot_general(x, gu[0:F], (((1,), (1,)), ((), ())),
                                   preferred_element_type=f32,
                                   precision=jax.lax.Precision.HIGHEST)
        up = jax.lax.dot_general(x, gu[F:2 * F], (((1,), (1,)), ((), ())),
                                 preferred_element_type=f32,
                                 precision=jax.lax.Precision.HIGHEST)
        act = gate * (1.0 / (1.0 + jnp.exp(-gate))) * up
        y = jax.lax.dot_general(act, dn_ref[0], (((1,), (1,)), ((), ())),
                                preferred_element_type=f32,
                                precision=jax.lax.Precision.HIGHEST)
        acc_ref[...] += jax.lax.dot_general(Ms, y, (((0,), (0,)), ((), ())),
                                            preferred_element_type=f32,
                                            precision=jax.lax.Precision.HIGHEST)

    @pl.when(b == NBMAX - 1)
    def _fin():
        out_ref[...] = acc_ref[...]


def kernel(hidden_states, router_weight, gate_up_proj, down_proj):
    Bv, Tv, Hv = hidden_states.shape
    E, H = router_weight.shape
    F = down_proj.shape[2]
    T = Bv * Tv
    K = 2
    NBMAX = T * K // TB + E - 1
    NBMAX = ((NBMAX + 7) // 8) * 8

    h2 = hidden_states.reshape(T, Hv)

    pwt, bexp8 = pl.pallas_call(
        functools.partial(_router_body, T=T, E=E, TB_=TB, NBMAX=NBMAX),
        out_shape=(
            jax.ShapeDtypeStruct((8, T), jnp.float32),
            jax.ShapeDtypeStruct((8, NBMAX), jnp.int32),
        ),
    )(h2, router_weight)

    bexp = bexp8[0]
    num = bexp8[1, 0:1]

    grid_spec = pltpu.PrefetchScalarGridSpec(
        num_scalar_prefetch=2,
        grid=(NBMAX,),
        in_specs=[
            pl.BlockSpec((8, T), lambda b, be, n: (0, 0)),
            pl.BlockSpec((T, Hv), lambda b, be, n: (0, 0)),
            pl.BlockSpec((1, 2 * F, H), lambda b, be, n: (be[b], 0, 0)),
            pl.BlockSpec((1, H, F), lambda b, be, n: (be[b], 0, 0)),
        ],
        out_specs=pl.BlockSpec((T, Hv), lambda b, be, n: (0, 0)),
        scratch_shapes=[pltpu.VMEM((T, Hv), jnp.float32)],
    )
    out = pl.pallas_call(
        functools.partial(_ffn_body, T=T, F=F, TB_=TB, NBMAX=NBMAX),
        grid_spec=grid_spec,
        out_shape=jax.ShapeDtypeStruct((T, Hv), jnp.float32),
        compiler_params=pltpu.CompilerParams(
            dimension_semantics=("arbitrary",)),
    )(bexp, num, pwt, h2, gate_up_proj, down_proj)

    return out.reshape(Bv, Tv, Hv)


# speedup vs baseline: 4.7131x; 4.7131x over previous
import functools

import jax
import jax.numpy as jnp
from jax.experimental import pallas as pl
from jax.experimental.pallas import tpu as pltpu

TB = 128


def _router_body(h_ref, rw_ref, pwt_ref, bexp_ref, *, T, E, TB_, NBMAX):
    f32 = jnp.float32
    bf16 = jnp.bfloat16
    h = h_ref[...]
    rw = rw_ref[...]
    logits = jax.lax.dot_general(h.astype(bf16), rw.astype(bf16),
                                 (((1,), (1,)), ((), ())),
                                 preferred_element_type=f32)
    lane = jax.lax.broadcasted_iota(jnp.int32, (T, E), 1).astype(f32)
    m1 = jnp.max(logits, axis=1, keepdims=True)
    i1 = jnp.min(jnp.where(logits == m1, lane, float(E)), axis=1, keepdims=True)
    e0 = (lane == i1).astype(f32)
    neg = jnp.where(lane == i1, -jnp.inf, logits)
    m2 = jnp.max(neg, axis=1, keepdims=True)
    i2 = jnp.min(jnp.where(neg == m2, lane, float(E)), axis=1, keepdims=True)
    e1 = (lane == i2).astype(f32)
    w0 = 1.0 / (1.0 + jnp.exp(m2 - m1))
    w1 = 1.0 - w0

    ones_col = jnp.ones((T, 1), f32)
    ecnt = jax.lax.dot_general(e0 + e1, ones_col, (((0,), (0,)), ((), ())),
                               preferred_element_type=f32)
    nblk = jnp.floor((ecnt + (TB_ - 1)) / TB_)
    tri = (jax.lax.broadcasted_iota(jnp.int32, (E, E), 1)
           < jax.lax.broadcasted_iota(jnp.int32, (E, E), 0)).astype(f32)
    excl = jax.lax.dot_general(tri, nblk, (((1,), (0,)), ((), ())),
                               preferred_element_type=f32)
    incl = excl + nblk

    Epairs = e0 + e1
    RB = 256
    c_blocks = []
    for tb in range(T // RB):
        r = jax.lax.broadcasted_iota(jnp.int32, (RB, T), 0) + tb * RB
        c = jax.lax.broadcasted_iota(jnp.int32, (RB, T), 1)
        lt = (c < r).astype(f32)
        c_blocks.append(jax.lax.dot_general(lt, Epairs, (((1,), (0,)), ((), ())),
                                            preferred_element_type=f32))
    C = jnp.concatenate(c_blocks, axis=0)

    blk0 = jax.lax.dot_general(e0, excl, (((1,), (0,)), ((), ())),
                               preferred_element_type=f32)
    blk1 = jax.lax.dot_general(e1, excl, (((1,), (0,)), ((), ())),
                               preferred_element_type=f32)
    rank0 = jnp.sum(C * e0, axis=1, keepdims=True)
    rank1 = jnp.sum(C * e1, axis=1, keepdims=True)
    pos0 = blk0 * TB_ + rank0
    pos1 = blk1 * TB_ + rank1

    cols = jnp.concatenate([pos0, pos1, w0, w1, w0, w0, w0, w0], axis=1)
    ident = (jax.lax.broadcasted_iota(jnp.int32, (T, T), 0)
             == jax.lax.broadcasted_iota(jnp.int32, (T, T), 1)).astype(f32)
    pwt_ref[...] = jax.lax.dot_general(cols, ident, (((0,), (0,)), ((), ())),
                                       preferred_element_type=f32,
                                       precision=jax.lax.Precision.HIGHEST)

    bidx = jax.lax.broadcasted_iota(jnp.int32, (E, NBMAX), 1).astype(f32)
    bexp = jnp.sum((incl <= bidx).astype(f32), axis=0, keepdims=True)
    bexp = jnp.minimum(bexp, float(E - 1))
    nused = jnp.sum(nblk)
    nrow = jnp.full((1, NBMAX), nused, f32)
    bexp_ref[...] = jnp.concatenate(
        [bexp, nrow, bexp, bexp, bexp, bexp, bexp, bexp], axis=0
    ).astype(jnp.int32)


def _ffn_body(bexp_sref, num_sref, pwt_ref, h_ref, gu_ref, dn_ref, out_ref,
              acc_ref, *, T, F, TB_, NBMAX):
    f32 = jnp.float32
    bf16 = jnp.bfloat16
    b = pl.program_id(0)

    @pl.when(b == 0)
    def _init():
        acc_ref[...] = jnp.zeros_like(acc_ref)

    @pl.when(b < num_sref[0])
    def _compute():
        pwt = pwt_ref[...]
        p0 = pwt[0:1, :]
        p1 = pwt[1:2, :]
        w0 = pwt[2:3, :]
        w1 = pwt[3:4, :]
        slot = (jax.lax.broadcasted_iota(jnp.int32, (TB_, T), 0)
                + b * TB_).astype(f32)
        m0 = (slot == p0).astype(bf16)
        m1 = (slot == p1).astype(bf16)
        Mg = m0 + m1
        Ms = (m0.astype(f32) * w0 + m1.astype(f32) * w1).astype(bf16)
        x = jax.lax.dot_general(Mg, h_ref[...].astype(bf16),
                                (((1,), (0,)), ((), ())),
                                preferred_element_type=f32)
        xb = x.astype(bf16)
        gu = gu_ref[0].astype(bf16)
        gate = jax.lax.dot_general(xb, gu[0:F], (((1,), (1,)), ((), ())),
                                   preferred_element_type=f32)
        up = jax.lax.dot_general(xb, gu[F:2 * F], (((1,), (1,)), ((), ())),
                                 preferred_element_type=f32)
        act = gate * (1.0 / (1.0 + jnp.exp(-gate))) * up
        y = jax.lax.dot_general(act.astype(bf16), dn_ref[0].astype(bf16),
                                (((1,), (1,)), ((), ())),
                                preferred_element_type=f32)
        acc_ref[...] += jax.lax.dot_general(Ms, y.astype(bf16),
                                            (((0,), (0,)), ((), ())),
                                            preferred_element_type=f32)

    @pl.when(b == NBMAX - 1)
    def _fin():
        out_ref[...] = acc_ref[...]


def kernel(hidden_states, router_weight, gate_up_proj, down_proj):
    Bv, Tv, Hv = hidden_states.shape
    E, H = router_weight.shape
    F = down_proj.shape[2]
    T = Bv * Tv
    K = 2
    NBMAX = T * K // TB + E - 1
    NBMAX = ((NBMAX + 7) // 8) * 8

    h2 = hidden_states.reshape(T, Hv)

    pwt, bexp8 = pl.pallas_call(
        functools.partial(_router_body, T=T, E=E, TB_=TB, NBMAX=NBMAX),
        out_shape=(
            jax.ShapeDtypeStruct((8, T), jnp.float32),
            jax.ShapeDtypeStruct((8, NBMAX), jnp.int32),
        ),
    )(h2, router_weight)

    bexp = bexp8[0]
    num = bexp8[1, 0:1]

    grid_spec = pltpu.PrefetchScalarGridSpec(
        num_scalar_prefetch=2,
        grid=(NBMAX,),
        in_specs=[
            pl.BlockSpec((8, T), lambda b, be, n: (0, 0)),
            pl.BlockSpec((T, Hv), lambda b, be, n: (0, 0)),
            pl.BlockSpec((1, 2 * F, H), lambda b, be, n: (be[b], 0, 0)),
            pl.BlockSpec((1, H, F), lambda b, be, n: (be[b], 0, 0)),
        ],
        out_specs=pl.BlockSpec((T, Hv), lambda b, be, n: (0, 0)),
        scratch_shapes=[pltpu.VMEM((T, Hv), jnp.float32)],
    )
    out = pl.pallas_call(
        functools.partial(_ffn_body, T=T, F=F, TB_=TB, NBMAX=NBMAX),
        grid_spec=grid_spec,
        out_shape=jax.ShapeDtypeStruct((T, Hv), jnp.float32),
        compiler_params=pltpu.CompilerParams(
            dimension_semantics=("arbitrary",)),
    )(bexp, num, pwt, h2, gate_up_proj, down_proj)

    return out.reshape(Bv, Tv, Hv)
